# baseline (speedup 1.0000x reference)
import functools

import jax
import jax.numpy as jnp
from jax import lax
from jax.experimental import pallas as pl
from jax.experimental.pallas import tpu as pltpu
from jax.experimental.pallas import tpu_sc as plsc

_PIECE_I = 4
_GSUB = 80


def kernel(context_mapping, table):
    n, l = context_mapping.shape
    d = table.shape[1]
    l2 = l // 2
    b2 = n * l2

    info = plsc.get_sparse_core_info()
    nw = info.num_cores * info.num_subcores
    lanes = info.num_lanes

    rows_pw = n // nw
    piece = _PIECE_I * l2
    n_pieces = rows_pw // _PIECE_I
    n_mul = piece // lanes
    n_gsub = piece // _GSUB

    t_even = table[0:l:2, :]
    t_odd = table[1:l:2, :]
    t_zero = jnp.broadcast_to(table[0:1, :], (l2, d))
    ptable = jnp.concatenate([
        jnp.concatenate([t_zero, t_zero], axis=1),
        jnp.concatenate([t_even, t_zero], axis=1),
        jnp.concatenate([t_zero, t_odd], axis=1),
        jnp.concatenate([t_even, t_odd], axis=1),
    ], axis=0)

    me_flat = context_mapping[:, 0::2].reshape(b2).astype(jnp.int32)
    mo_flat = context_mapping[:, 1::2].reshape(b2).astype(jnp.int32)
    jseq = jnp.tile(jnp.arange(l2, dtype=jnp.int32), _PIECE_I)

    @functools.partial(
        pl.kernel,
        mesh=plsc.VectorSubcoreMesh(core_axis_name="c", subcore_axis_name="s"),
        out_type=jax.ShapeDtypeStruct((b2, 2 * d), jnp.float32),
        scratch_types=[
            pltpu.VMEM((piece,), jnp.int32),
            pltpu.VMEM((piece,), jnp.int32),
            pltpu.VMEM((piece,), jnp.int32),
            pltpu.VMEM((piece,), jnp.int32),
            pltpu.VMEM((piece, 2 * d), jnp.float32),
            pltpu.SemaphoreType.DMA,
        ],
    )
    def sc_gather(me_hbm, mo_hbm, jseq_hbm, ptable_hbm, out_hbm,
                  jseq_v, me_v, mo_v, idx_v, rows_v, sem):
        wid = lax.axis_index("s") * info.num_cores + lax.axis_index("c")
        base = wid * rows_pw * l2
        pltpu.sync_copy(jseq_hbm, jseq_v)

        def body(t, carry):
            off = pl.multiple_of(base + t * piece, piece)
            pltpu.sync_copy(me_hbm.at[pl.ds(off, piece)], me_v)
            pltpu.sync_copy(mo_hbm.at[pl.ds(off, piece)], mo_v)
            for v in range(n_mul):
                sl = pl.ds(v * lanes, lanes)
                idx_v[sl] = (me_v[sl] + 2 * mo_v[sl]) * l2 + jseq_v[sl]
            copies = []
            for g in range(n_gsub):
                gs = pl.ds(g * _GSUB, _GSUB)
                copies.append(pltpu.async_copy(
                    ptable_hbm.at[idx_v.at[gs]], rows_v.at[gs], sem))
            for c in copies:
                c.wait()
            pltpu.sync_copy(rows_v, out_hbm.at[pl.ds(off, piece)])
            return carry

        lax.fori_loop(0, n_pieces, body, 0)

    out = sc_gather(me_flat, mo_flat, jseq, ptable)
    return out.reshape(n, l, d)
